# baseline (speedup 1.0000x reference)
import jax
import jax.numpy as jnp
import numpy as np
from jax import lax
from jax.experimental import pallas as pl
from jax.experimental.pallas import tpu as pltpu
from jax.experimental.pallas import tpu_sc as plsc

N = 65536
D = 32
STEP = np.float32(1.0 / N)
NC = 2
NS = 16
NW = NC * NS
B_PER_W = N // NW
L = 16
CHUNK = 128
PHASE = 512
NPHASE = B_PER_W // PHASE
CPP = PHASE // CHUNK


def _resample_body(cum_hbm, off_hbm, table_hbm, out_hbm,
                   cum_v, off_v, idx_v, rows_v, gsems, wsems):
    wid = lax.axis_index("s") * NC + lax.axis_index("c")
    base = wid * B_PER_W

    pltpu.sync_copy(cum_hbm, cum_v)
    pltpu.sync_copy(off_hbm, off_v)
    off = off_v[...]
    lanes = lax.iota(jnp.int32, L)

    def search_chunk(c):
        for q in range(CHUNK // L):
            jv = base + c * CHUNK + q * L + lanes
            pos = off + STEP * jv.astype(jnp.float32)
            r = jnp.zeros((L,), jnp.int32)
            s = 1 << 15
            while s >= 1:
                t = r + s
                cm = plsc.load_gather(cum_v, [t - 1])
                r = jnp.where(cm < pos, t, r)
                s >>= 1
            idx_v[pl.ds(c * CHUNK + q * L, L)] = jnp.minimum(r, N - 1)

    wb: list = [None, None]
    for p in range(NPHASE):
        buf = rows_v.at[pl.ds((p % 2) * PHASE, PHASE)]
        if wb[p % 2] is not None:
            wb[p % 2].wait()

        def phase_chunk(c, carry, p=p, buf=buf):
            search_chunk(c)
            pltpu.async_copy(
                table_hbm.at[idx_v.at[pl.ds(c * CHUNK, CHUNK)]],
                buf.at[pl.ds((c - p * CPP) * CHUNK, CHUNK)],
                gsems.at[p % 2],
            )
            return carry

        lax.fori_loop(p * CPP, (p + 1) * CPP, phase_chunk, 0)
        pltpu.make_async_copy(table_hbm.at[pl.ds(0, PHASE)], buf,
                              gsems.at[p % 2]).wait()
        wb[p % 2] = pltpu.async_copy(
            buf, out_hbm.at[pl.ds(base + p * PHASE, PHASE)], wsems.at[p % 2])
    wb[0].wait()
    wb[1].wait()


def _sc_resample(cum, off_arr, particles):
    run = pl.kernel(
        _resample_body,
        out_type=jax.ShapeDtypeStruct((N, D), jnp.float32),
        mesh=plsc.VectorSubcoreMesh(core_axis_name="c", subcore_axis_name="s"),
        scratch_types=[
            pltpu.VMEM((N,), jnp.float32),
            pltpu.VMEM((L,), jnp.float32),
            pltpu.VMEM((B_PER_W,), jnp.int32),
            pltpu.VMEM((2 * PHASE, D), jnp.float32),
            pltpu.SemaphoreType.DMA((2,)),
            pltpu.SemaphoreType.DMA((2,)),
        ],
        compiler_params=pltpu.CompilerParams(use_tc_tiling_on_sc=False,
                                             needs_layout_passes=False),
    )
    return run(cum, off_arr, particles)


def kernel(particles, particles_probs):
    n = particles.shape[0]
    probs = particles_probs / jnp.sum(particles_probs)
    cum = jnp.cumsum(probs)
    rnd_offset = jax.random.uniform(jax.random.key(42), (), dtype=jnp.float32,
                                    minval=0.0, maxval=1.0 / n)
    off_arr = jnp.full((L,), rnd_offset, dtype=jnp.float32)
    return _sc_resample(cum, off_arr, particles)


# speedup vs baseline: 1.0122x; 1.0122x over previous
import jax
import jax.numpy as jnp
import numpy as np
from jax import lax
from jax.experimental import pallas as pl
from jax.experimental.pallas import tpu as pltpu
from jax.experimental.pallas import tpu_sc as plsc

N = 65536
D = 32
STEP = np.float32(1.0 / N)
NC = 2
NS = 16
NW = NC * NS
B_PER_W = N // NW
L = 16
CHUNK = 128
PHASE = 512
NPHASE = B_PER_W // PHASE
CPP = PHASE // CHUNK


def _resample_body(cum_hbm, off_hbm, table_hbm, out_hbm,
                   cum_v, off_v, idx_v, rows_v, gsems, wsems):
    wid = lax.axis_index("s") * NC + lax.axis_index("c")
    base = wid * B_PER_W

    pltpu.sync_copy(cum_hbm, cum_v)
    pltpu.sync_copy(off_hbm, off_v)
    off = off_v[...]
    lanes = lax.iota(jnp.int32, L)

    def full_search_chunk(c):
        for q in range(CHUNK // L):
            jv = base + c * CHUNK + q * L + lanes
            pos = off + STEP * jv.astype(jnp.float32)
            r = jnp.zeros((L,), jnp.int32)
            s = 1 << 15
            while s >= 1:
                t = r + s
                cm = plsc.load_gather(cum_v, [t - 1])
                r = jnp.where(cm < pos, t, r)
                s >>= 1
            idx_v[pl.ds(c * CHUNK + q * L, L)] = jnp.minimum(r, N - 1)

    def search_chunk(c, lo_prev):
        bad = jnp.zeros((L,), jnp.int32)
        lo = lo_prev
        for q in range(CHUNK // L):
            jv = base + c * CHUNK + q * L + lanes
            pos = off + STEP * jv.astype(jnp.float32)
            r = jnp.full((L,), lo, jnp.int32)
            s = 64
            while s >= 1:
                t = r + s
                cm = plsc.load_gather(cum_v, [jnp.minimum(t - 1, N - 1)])
                r = jnp.where((t <= N) & (cm < pos), t, r)
                s >>= 1
            cmr = plsc.load_gather(cum_v, [jnp.minimum(r, N - 1)])
            bad = bad | jnp.where((r < N) & (cmr < pos), 1, 0)
            idx_v[pl.ds(c * CHUNK + q * L, L)] = jnp.minimum(r, N - 1)
            lo = jnp.max(r)
        lax.cond(jnp.max(bad) > 0, lambda: full_search_chunk(c), lambda: None)
        return jnp.max(idx_v[pl.ds(c * CHUNK + CHUNK - L, L)])

    wb: list = [None, None]
    lo_carry = jnp.int32(0)
    for p in range(NPHASE):
        buf = rows_v.at[pl.ds((p % 2) * PHASE, PHASE)]
        if wb[p % 2] is not None:
            wb[p % 2].wait()

        def phase_chunk(c, carry, p=p, buf=buf):
            lo_next = search_chunk(c, carry)
            pltpu.async_copy(
                table_hbm.at[idx_v.at[pl.ds(c * CHUNK, CHUNK)]],
                buf.at[pl.ds((c - p * CPP) * CHUNK, CHUNK)],
                gsems.at[p % 2],
            )
            return lo_next

        lo_carry = lax.fori_loop(p * CPP, (p + 1) * CPP, phase_chunk, lo_carry)
        pltpu.make_async_copy(table_hbm.at[pl.ds(0, PHASE)], buf,
                              gsems.at[p % 2]).wait()
        wb[p % 2] = pltpu.async_copy(
            buf, out_hbm.at[pl.ds(base + p * PHASE, PHASE)], wsems.at[p % 2])
    wb[0].wait()
    wb[1].wait()


def _sc_resample(cum, off_arr, particles):
    run = pl.kernel(
        _resample_body,
        out_type=jax.ShapeDtypeStruct((N, D), jnp.float32),
        mesh=plsc.VectorSubcoreMesh(core_axis_name="c", subcore_axis_name="s"),
        scratch_types=[
            pltpu.VMEM((N,), jnp.float32),
            pltpu.VMEM((L,), jnp.float32),
            pltpu.VMEM((B_PER_W,), jnp.int32),
            pltpu.VMEM((2 * PHASE, D), jnp.float32),
            pltpu.SemaphoreType.DMA((2,)),
            pltpu.SemaphoreType.DMA((2,)),
        ],
        compiler_params=pltpu.CompilerParams(use_tc_tiling_on_sc=False,
                                             needs_layout_passes=False),
    )
    return run(cum, off_arr, particles)


def kernel(particles, particles_probs):
    n = particles.shape[0]
    probs = particles_probs / jnp.sum(particles_probs)
    cum = jnp.cumsum(probs)
    rnd_offset = jax.random.uniform(jax.random.key(42), (), dtype=jnp.float32,
                                    minval=0.0, maxval=1.0 / n)
    off_arr = jnp.full((L,), rnd_offset, dtype=jnp.float32)
    return _sc_resample(cum, off_arr, particles)
